# baseline (speedup 1.0000x reference)
import jax
import jax.numpy as jnp
from jax import lax
from jax.experimental import pallas as pl
from jax.experimental.pallas import tpu as pltpu
from jax.experimental.pallas import tpu_sc as plsc

NUM_CORES = 2
NUM_SUBCORES = 16
NW = NUM_CORES * NUM_SUBCORES
LANES = 16

BATCH = 16384
EMBED_DIM = 32
NUM_ROWS = 1000000
B_PER_W = BATCH // NW
CHUNK = 8
N_CHUNKS = B_PER_W // CHUNK
RING = 16


def _gather_kernel(su_hbm, sb_hbm, table_t_hbm, staged_hbm,
                   su_v, sb_v, ring_v, temp_v, pad_v, fsem, ssem):
    wid = lax.axis_index("s") * NUM_CORES + lax.axis_index("c")

    pltpu.sync_copy(su_hbm.at[wid, 0], su_v.at[pl.ds(0, B_PER_W)])
    pltpu.sync_copy(sb_hbm.at[wid, 0], sb_v.at[pl.ds(0, B_PER_W)])

    d_lo = lax.iota(jnp.int32, LANES)

    def chunk_body(m, carry):
        slot_base, prev_blk = carry
        c0 = pl.multiple_of(m * CHUNK, CHUNK)
        uv = su_v[pl.ds(c0, LANES)]
        bv = sb_v[pl.ds(c0, LANES)]

        slots = []
        n_new = jnp.int32(0)
        prev = prev_blk
        slot = slot_base
        for k in range(CHUNK):
            blk = uv[k] >> 7
            is_new = blk != prev
            slot = slot + is_new.astype(jnp.int32)
            n_new = n_new + is_new.astype(jnp.int32)

            @pl.when(is_new)
            def _(blk=blk, slot=slot):
                o = pl.multiple_of(blk * 128, 128)
                pltpu.async_copy(table_t_hbm.at[:, pl.ds(o, 128)],
                                 ring_v.at[slot & (RING - 1)], fsem)

            slots.append(slot)
            prev = blk

        def drain_fetch(_, c):
            pltpu.make_async_copy(table_t_hbm.at[:, pl.ds(0, 128)],
                                  ring_v.at[0], fsem).wait()
            return c

        lax.fori_loop(0, n_new, drain_fetch, 0)

        for k in range(CHUNK):
            c = jnp.full((LANES,), uv[k] & 127, jnp.int32)
            ss = jnp.full((LANES,), slots[k] & (RING - 1), jnp.int32)
            e0 = plsc.load_gather(ring_v, [ss, d_lo, c])
            e1 = plsc.load_gather(ring_v, [ss, d_lo + LANES, c])
            temp_v[k, pl.ds(0, LANES)] = e0
            temp_v[k, pl.ds(LANES, LANES)] = e1
            boff = bv[k] * EMBED_DIM
            pltpu.async_copy(
                temp_v.at[k],
                staged_hbm.at[pl.ds(pl.multiple_of(boff, CHUNK), EMBED_DIM)],
                ssem)

        def drain_scatter(_, c):
            pltpu.make_async_copy(temp_v.at[0],
                                  staged_hbm.at[pl.ds(0, EMBED_DIM)],
                                  ssem).wait()
            return c

        lax.fori_loop(0, CHUNK, drain_scatter, 0)
        return slot, prev

    lax.fori_loop(0, N_CHUNKS, chunk_body,
                  (jnp.int32(0), jnp.int32(-1)))

    @pl.when(wid == 0)
    def _():
        pad_v[pl.ds(0, LANES)] = jnp.zeros((LANES,), jnp.float32)
        pltpu.sync_copy(pad_v,
                        staged_hbm.at[pl.ds(BATCH * EMBED_DIM, LANES)])


def _dot_kernel(us_hbm, is_hbm, out_hbm, uvals_v, ivals_v, out_v, sem):
    wid = lax.axis_index("s") * NUM_CORES + lax.axis_index("c")
    base = wid * B_PER_W

    cp1 = pltpu.async_copy(
        us_hbm.at[pl.ds(base * EMBED_DIM, B_PER_W * EMBED_DIM)], uvals_v, sem)
    cp2 = pltpu.async_copy(
        is_hbm.at[pl.ds(base * EMBED_DIM, B_PER_W * EMBED_DIM)], ivals_v, sem)
    cp1.wait()
    cp2.wait()

    lane32 = lax.iota(jnp.int32, LANES) * EMBED_DIM

    def group_body(g, _):
        flat0 = lane32 + g * (LANES * EMBED_DIM)
        acc = jnp.zeros((LANES,), jnp.float32)
        for j in range(EMBED_DIM):
            fj = flat0 + j
            uv = plsc.load_gather(uvals_v, [fj])
            iv = plsc.load_gather(ivals_v, [fj])
            acc = acc + uv * iv
        out_v[pl.ds(pl.multiple_of(g * LANES, LANES), LANES)] = acc
        return 0

    lax.fori_loop(0, B_PER_W // LANES, group_body, 0)

    pltpu.sync_copy(out_v, out_hbm.at[pl.ds(base, B_PER_W)])


@jax.jit
def _mf_dot(user_indices, item_indices, user_table, item_table):
    mesh = plsc.VectorSubcoreMesh(core_axis_name="c", subcore_axis_name="s")
    params = pltpu.CompilerParams(
        needs_layout_passes=False, use_tc_tiling_on_sc=True)

    gather_fn = pl.kernel(
        _gather_kernel,
        out_type=jax.ShapeDtypeStruct((BATCH * EMBED_DIM + LANES,),
                                      jnp.float32),
        mesh=mesh,
        compiler_params=params,
        scratch_types=[
            pltpu.VMEM((B_PER_W + LANES,), jnp.int32),
            pltpu.VMEM((B_PER_W + LANES,), jnp.int32),
            pltpu.VMEM((RING, EMBED_DIM, 128), jnp.float32),
            pltpu.VMEM((CHUNK, EMBED_DIM), jnp.float32),
            pltpu.VMEM((LANES,), jnp.float32),
            pltpu.SemaphoreType.DMA,
            pltpu.SemaphoreType.DMA,
        ],
    )

    dot_fn = pl.kernel(
        _dot_kernel,
        out_type=jax.ShapeDtypeStruct((BATCH,), jnp.float32),
        mesh=mesh,
        compiler_params=params,
        scratch_types=[
            pltpu.VMEM((B_PER_W * EMBED_DIM,), jnp.float32),
            pltpu.VMEM((B_PER_W * EMBED_DIM,), jnp.float32),
            pltpu.VMEM((B_PER_W,), jnp.float32),
            pltpu.SemaphoreType.DMA,
        ],
    )

    def sorted_stream(idx):
        idx = idx.astype(jnp.int32)
        perm = jnp.argsort(idx)
        su = idx[perm].reshape(NW, 1, B_PER_W)
        sb = perm.astype(jnp.int32).reshape(NW, 1, B_PER_W)
        return su, sb

    su, sbu = sorted_stream(user_indices)
    si, sbi = sorted_stream(item_indices)

    u_staged = gather_fn(su, sbu, user_table.T)
    i_staged = gather_fn(si, sbi, item_table.T)
    return dot_fn(u_staged, i_staged)


def kernel(user_indices, item_indices, user_table, item_table):
    return _mf_dot(user_indices, item_indices, user_table, item_table)


# speedup vs baseline: 1.1910x; 1.1910x over previous
import jax
import jax.numpy as jnp
from jax import lax
from jax.experimental import pallas as pl
from jax.experimental.pallas import tpu as pltpu
from jax.experimental.pallas import tpu_sc as plsc

NUM_CORES = 2
NUM_SUBCORES = 16
NW = NUM_CORES * NUM_SUBCORES
LANES = 16

BATCH = 16384
EMBED_DIM = 32
NUM_ROWS = 1000000
B_PER_W = BATCH // NW
CHUNK = 16
N_CHUNKS = B_PER_W // CHUNK
RING = 24


def _gather_kernel(su_hbm, sb_hbm, table_t_hbm, staged_hbm,
                   su_v, sb_v, ring_v, temp_v, pad_v, fsem, ssem):
    wid = lax.axis_index("s") * NUM_CORES + lax.axis_index("c")

    pltpu.sync_copy(su_hbm.at[wid, 0], su_v.at[pl.ds(0, B_PER_W)])
    pltpu.sync_copy(sb_hbm.at[wid, 0], sb_v.at[pl.ds(0, B_PER_W)])

    d_lo = lax.iota(jnp.int32, LANES)

    def chunk_body(m, carry):
        slot_base, prev_blk = carry
        c0 = pl.multiple_of(m * CHUNK, CHUNK)
        uv = su_v[pl.ds(c0, LANES)]
        bv = sb_v[pl.ds(c0, LANES)]

        slots = []
        n_new = jnp.int32(0)
        prev = prev_blk
        slot = slot_base
        for k in range(CHUNK):
            blk = uv[k] >> 7
            is_new = blk != prev
            slot = slot + is_new.astype(jnp.int32)
            n_new = n_new + is_new.astype(jnp.int32)

            @pl.when(is_new)
            def _(blk=blk, slot=slot):
                o = pl.multiple_of(blk * 128, 128)
                pltpu.async_copy(table_t_hbm.at[:, pl.ds(o, 128)],
                                 ring_v.at[slot % RING], fsem)

            slots.append(slot)
            prev = blk

        def drain_fetch(_, c):
            pltpu.make_async_copy(table_t_hbm.at[:, pl.ds(0, 128)],
                                  ring_v.at[0], fsem).wait()
            return c

        lax.fori_loop(0, n_new, drain_fetch, 0)

        for k in range(CHUNK):
            c = jnp.full((LANES,), uv[k] & 127, jnp.int32)
            ss = jnp.full((LANES,), slots[k] % RING, jnp.int32)
            e0 = plsc.load_gather(ring_v, [ss, d_lo, c])
            e1 = plsc.load_gather(ring_v, [ss, d_lo + LANES, c])
            temp_v[k, pl.ds(0, LANES)] = e0
            temp_v[k, pl.ds(LANES, LANES)] = e1
            boff = bv[k] * EMBED_DIM
            pltpu.async_copy(
                temp_v.at[k],
                staged_hbm.at[pl.ds(pl.multiple_of(boff, CHUNK), EMBED_DIM)],
                ssem)

        def drain_scatter(_, c):
            pltpu.make_async_copy(temp_v.at[0],
                                  staged_hbm.at[pl.ds(0, EMBED_DIM)],
                                  ssem).wait()
            return c

        lax.fori_loop(0, CHUNK, drain_scatter, 0)
        return slot, prev

    lax.fori_loop(0, N_CHUNKS, chunk_body,
                  (jnp.int32(0), jnp.int32(-1)))

    @pl.when(wid == 0)
    def _():
        pad_v[pl.ds(0, LANES)] = jnp.zeros((LANES,), jnp.float32)
        pltpu.sync_copy(pad_v,
                        staged_hbm.at[pl.ds(BATCH * EMBED_DIM, LANES)])


def _dot_kernel(us_hbm, is_hbm, out_hbm, uvals_v, ivals_v, out_v, sem):
    wid = lax.axis_index("s") * NUM_CORES + lax.axis_index("c")
    base = wid * B_PER_W

    cp1 = pltpu.async_copy(
        us_hbm.at[pl.ds(base * EMBED_DIM, B_PER_W * EMBED_DIM)], uvals_v, sem)
    cp2 = pltpu.async_copy(
        is_hbm.at[pl.ds(base * EMBED_DIM, B_PER_W * EMBED_DIM)], ivals_v, sem)
    cp1.wait()
    cp2.wait()

    lane32 = lax.iota(jnp.int32, LANES) * EMBED_DIM

    def group_body(g, _):
        flat0 = lane32 + g * (LANES * EMBED_DIM)
        acc = jnp.zeros((LANES,), jnp.float32)
        for j in range(EMBED_DIM):
            fj = flat0 + j
            uv = plsc.load_gather(uvals_v, [fj])
            iv = plsc.load_gather(ivals_v, [fj])
            acc = acc + uv * iv
        out_v[pl.ds(pl.multiple_of(g * LANES, LANES), LANES)] = acc
        return 0

    lax.fori_loop(0, B_PER_W // LANES, group_body, 0)

    pltpu.sync_copy(out_v, out_hbm.at[pl.ds(base, B_PER_W)])


@jax.jit
def _mf_dot(user_indices, item_indices, user_table, item_table):
    mesh = plsc.VectorSubcoreMesh(core_axis_name="c", subcore_axis_name="s")
    params = pltpu.CompilerParams(
        needs_layout_passes=False, use_tc_tiling_on_sc=True)

    gather_fn = pl.kernel(
        _gather_kernel,
        out_type=jax.ShapeDtypeStruct((BATCH * EMBED_DIM + LANES,),
                                      jnp.float32),
        mesh=mesh,
        compiler_params=params,
        scratch_types=[
            pltpu.VMEM((B_PER_W + LANES,), jnp.int32),
            pltpu.VMEM((B_PER_W + LANES,), jnp.int32),
            pltpu.VMEM((RING, EMBED_DIM, 128), jnp.float32),
            pltpu.VMEM((CHUNK, EMBED_DIM), jnp.float32),
            pltpu.VMEM((LANES,), jnp.float32),
            pltpu.SemaphoreType.DMA,
            pltpu.SemaphoreType.DMA,
        ],
    )

    dot_fn = pl.kernel(
        _dot_kernel,
        out_type=jax.ShapeDtypeStruct((BATCH,), jnp.float32),
        mesh=mesh,
        compiler_params=params,
        scratch_types=[
            pltpu.VMEM((B_PER_W * EMBED_DIM,), jnp.float32),
            pltpu.VMEM((B_PER_W * EMBED_DIM,), jnp.float32),
            pltpu.VMEM((B_PER_W,), jnp.float32),
            pltpu.SemaphoreType.DMA,
        ],
    )

    def sorted_stream(idx):
        idx = idx.astype(jnp.int32)
        perm = jnp.argsort(idx)
        su = idx[perm].reshape(NW, 1, B_PER_W)
        sb = perm.astype(jnp.int32).reshape(NW, 1, B_PER_W)
        return su, sb

    su, sbu = sorted_stream(user_indices)
    si, sbi = sorted_stream(item_indices)

    u_staged = gather_fn(su, sbu, user_table.T)
    i_staged = gather_fn(si, sbi, item_table.T)
    return dot_fn(u_staged, i_staged)


def kernel(user_indices, item_indices, user_table, item_table):
    return _mf_dot(user_indices, item_indices, user_table, item_table)
